# Initial kernel scaffold; baseline (speedup 1.0000x reference)
import jax
import jax.numpy as jnp
from jax.experimental import pallas as pl


def kernel(node_des, edge_index, atomtype, des_radial_hop, W_env, W_env_adj, b_env, W_edge, W_edge_adj, b_edge):
    raise NotImplementedError("write your pallas kernel here")


# speedup vs baseline: 21.2208x; 21.2208x over previous
import dataclasses
import functools

import jax
import jax.numpy as jnp
import numpy as np
from jax import lax
from jax.experimental import pallas as pl
from jax.experimental.pallas import tpu as pltpu
from jax.experimental.pallas import tpu_sc as plsc

N = 10000
E = 320000
NMAX = 16
NOUT = 10
NBOND = 6

SC_NC = 2
SC_NS = 16
SC_NW = SC_NC * SC_NS
WIN = 128
NWIN = E // WIN
WIN_PER_SUB = NWIN // SC_NW
WIN_REM = NWIN - WIN_PER_SUB * SC_NW

ROWS = E // 8
RB = 800
NCHUNK = 128

_P16_NP = np.zeros((8, 128), dtype=np.float32)
for _e in range(8):
    _P16_NP[_e, _e * 16:(_e + 1) * 16] = 1.0

_C_NP = np.zeros((128, 80), dtype=np.float32)
for _e in range(8):
    for _o in range(NOUT):
        _C_NP[_e * 16 + _o, _e * 10 + _o] = 1.0


def _sc_body(nd_hbm, ih_hbm, jh_hbm, at_hbm, des_hbm, bond_hbm,
             at_v, idxi_v, idxj_v, gi_v, gj_v, bond_v, sem1, sem2):
    wid = lax.axis_index("s") * SC_NC + lax.axis_index("c")
    pltpu.sync_copy(at_hbm, at_v)

    def do_window(widx):
        b0 = widx * WIN
        pltpu.sync_copy(ih_hbm.at[pl.ds(b0, WIN)], idxi_v)
        pltpu.sync_copy(jh_hbm.at[pl.ds(b0, WIN)], idxj_v)
        cp1 = pltpu.async_copy(nd_hbm.at[idxi_v], gi_v, sem1)
        cp2 = pltpu.async_copy(nd_hbm.at[idxj_v], gj_v, sem2)

        @pl.loop(0, WIN, step=16)
        def _(g):
            iv = idxi_v[pl.ds(g, 16)]
            jv = idxj_v[pl.ds(g, 16)]
            ti = plsc.load_gather(at_v, [iv])
            tj = plsc.load_gather(at_v, [jv])
            s = ti * ti + tj * tj
            bnd = s - jnp.where(s >= 3, 1, 0) - jnp.where(s >= 6, 2, 0)
            bond_v[pl.ds(g, 16)] = bnd.astype(jnp.float32)

        cp1.wait()
        cp2.wait()

        @pl.loop(0, WIN)
        def _(e):
            gi_v[e, :] = gi_v[e, :] + gj_v[e, :]

        pltpu.sync_copy(gi_v, des_hbm.at[pl.ds(b0, WIN)])
        pltpu.sync_copy(bond_v, bond_hbm.at[pl.ds(b0, WIN)])

    @pl.loop(0, WIN_PER_SUB)
    def _(k):
        do_window(k * SC_NW + wid)

    @pl.when(wid < WIN_REM)
    def _():
        do_window(WIN_PER_SUB * SC_NW + wid)


@jax.jit
def _sc_gather(node_des, i_hop, j_hop, atomtype):
    mesh = plsc.VectorSubcoreMesh(core_axis_name="c", subcore_axis_name="s")
    cp = pltpu.CompilerParams(use_tc_tiling_on_sc=False)
    if "needs_layout_passes" in pltpu.CompilerParams.__dataclass_fields__:
        cp = dataclasses.replace(cp, needs_layout_passes=False)
    run = pl.kernel(
        _sc_body,
        out_type=(
            jax.ShapeDtypeStruct((E, NMAX), jnp.float32),
            jax.ShapeDtypeStruct((E,), jnp.float32),
        ),
        mesh=mesh,
        scratch_types=[
            pltpu.VMEM((N,), jnp.int32),
            pltpu.VMEM((WIN,), jnp.int32),
            pltpu.VMEM((WIN,), jnp.int32),
            pltpu.VMEM((WIN, NMAX), jnp.float32),
            pltpu.VMEM((WIN, NMAX), jnp.float32),
            pltpu.VMEM((WIN,), jnp.float32),
            pltpu.SemaphoreType.DMA,
            pltpu.SemaphoreType.DMA,
        ],
        compiler_params=cp,
    )
    return run(node_des, i_hop, j_hop, atomtype)


def _tc_body(desp_ref, drhp_ref, bondp_ref, kenv_ref, kedge_ref, p16_ref,
             cmat_ref, benv_ref, bedge_ref, out_ref):
    desp = desp_ref[...]
    b16 = jnp.dot(bondp_ref[...], p16_ref[...],
                  preferred_element_type=jnp.float32)
    env_all = jnp.dot(desp, kenv_ref[...],
                      preferred_element_type=jnp.float32)
    env_sel = jnp.where(b16 == 0.0, env_all[:, 0:NCHUNK], 0.0)
    for b in range(1, NBOND):
        env_sel = env_sel + jnp.where(
            b16 == float(b), env_all[:, b * NCHUNK:(b + 1) * NCHUNK], 0.0)
    env = jnp.tanh(env_sel) + env_all[:, NBOND * NCHUNK:] + benv_ref[...]
    x = env * drhp_ref[...]
    hop_all = jnp.dot(x, kedge_ref[...],
                      preferred_element_type=jnp.float32)
    hop_sel = jnp.where(b16 == 0.0, hop_all[:, 0:NCHUNK], 0.0)
    for b in range(1, NBOND):
        hop_sel = hop_sel + jnp.where(
            b16 == float(b), hop_all[:, b * NCHUNK:(b + 1) * NCHUNK], 0.0)
    hop = jnp.tanh(hop_sel) + hop_all[:, NBOND * NCHUNK:] + bedge_ref[...]
    out_ref[...] = jnp.dot(hop, cmat_ref[...],
                           preferred_element_type=jnp.float32)


def _tc_dense(desp, drhp, bondp, kenv, kedge, p16, cmat, benv, bedge):
    full = lambda shape: pl.BlockSpec(shape, lambda i: (0, 0))
    return pl.pallas_call(
        _tc_body,
        grid=(ROWS // RB,),
        in_specs=[
            pl.BlockSpec((RB, 128), lambda i: (i, 0)),
            pl.BlockSpec((RB, 128), lambda i: (i, 0)),
            pl.BlockSpec((RB, 8), lambda i: (i, 0)),
            full((128, (NBOND + 1) * NCHUNK)),
            full((128, (NBOND + 1) * NCHUNK)),
            full((8, 128)),
            full((128, 80)),
            full((1, 128)),
            full((1, 128)),
        ],
        out_specs=pl.BlockSpec((RB, 80), lambda i: (i, 0)),
        out_shape=jax.ShapeDtypeStruct((ROWS, 80), jnp.float32),
    )(desp, drhp, bondp, kenv, kedge, p16, cmat, benv, bedge)


def kernel(node_des, edge_index, atomtype, des_radial_hop,
           W_env, W_env_adj, b_env, W_edge, W_edge_adj, b_edge):
    i_hop = edge_index[0]
    j_hop = edge_index[1]
    des, bond = _sc_gather(node_des, i_hop, j_hop, atomtype)

    eye8 = jnp.eye(8, dtype=jnp.float32)
    kron8 = lambda w: jnp.kron(eye8, w)
    kenv = jnp.concatenate(
        [kron8(W_env[b].T) for b in range(NBOND)] + [kron8(W_env_adj.T)],
        axis=1)
    wep = jnp.pad(jnp.swapaxes(W_edge, 1, 2), ((0, 0), (0, 0), (0, NMAX - NOUT)))
    kedge = jnp.concatenate(
        [kron8(wep[b]) for b in range(NBOND)]
        + [kron8(jnp.pad(W_edge_adj.T, ((0, 0), (0, NMAX - NOUT))))],
        axis=1)
    benv128 = jnp.tile(b_env, 8)[None, :]
    bedge128 = jnp.tile(jnp.pad(b_edge, (0, NMAX - NOUT)), 8)[None, :]

    desp = des.reshape(ROWS, 128)
    drhp = des_radial_hop.reshape(ROWS, 128)
    bondp = bond.reshape(ROWS, 8)
    outp = _tc_dense(desp, drhp, bondp, kenv, kedge,
                     jnp.asarray(_P16_NP), jnp.asarray(_C_NP),
                     benv128, bedge128)
    return outp.reshape(E, NOUT)


# speedup vs baseline: 23.1684x; 1.0918x over previous
import dataclasses
import functools

import jax
import jax.numpy as jnp
import numpy as np
from jax import lax
from jax.experimental import pallas as pl
from jax.experimental.pallas import tpu as pltpu
from jax.experimental.pallas import tpu_sc as plsc

N = 10000
E = 320000
NMAX = 16
NOUT = 10
NBOND = 6

SC_NC = 2
SC_NS = 16
SC_NW = SC_NC * SC_NS
WIN = 128
NWIN = E // WIN
WIN_PER_SUB = NWIN // SC_NW
WIN_REM = NWIN - WIN_PER_SUB * SC_NW

ROWS = E // 8
RB = 800
NCHUNK = 128

_P16_NP = np.zeros((8, 128), dtype=np.float32)
for _e in range(8):
    _P16_NP[_e, _e * 16:(_e + 1) * 16] = 1.0

_C_NP = np.zeros((128, 80), dtype=np.float32)
for _e in range(8):
    for _o in range(NOUT):
        _C_NP[_e * 16 + _o, _e * 10 + _o] = 1.0


PACK = WIN // 8


def _sc_body(nd_hbm, ih_hbm, jh_hbm, at_hbm, des_hbm, b16_hbm,
             at_v, bond_v,
             idxi0, idxj0, gi0, gj0, desw0, bw0,
             idxi1, idxj1, gi1, gj1, desw1, bw1,
             sg0, sg1, so0, so1):
    wid = lax.axis_index("s") * SC_NC + lax.axis_index("c")
    pltpu.sync_copy(at_hbm, at_v)

    bufs = ((idxi0, idxj0, gi0, gj0, desw0, bw0, sg0, so0),
            (idxi1, idxj1, gi1, gj1, desw1, bw1, sg1, so1))

    def fire(widx, b):
        idxi, idxj, gi, gj, _, _, sg, _ = bufs[b]
        b0 = widx * WIN
        pltpu.sync_copy(ih_hbm.at[pl.ds(b0, WIN)], idxi)
        pltpu.sync_copy(jh_hbm.at[pl.ds(b0, WIN)], idxj)
        pltpu.async_copy(nd_hbm.at[idxi], gi, sg)
        pltpu.async_copy(nd_hbm.at[idxj], gj, sg)

    def compute(widx, b, outwait):
        idxi, idxj, gi, gj, desw, bw, sg, so = bufs[b]
        pltpu.make_async_copy(nd_hbm.at[idxi], gi, sg).wait()
        pltpu.make_async_copy(nd_hbm.at[idxj], gj, sg).wait()

        @pl.loop(0, WIN, step=16)
        def _(g):
            iv = idxi[pl.ds(g, 16)]
            jv = idxj[pl.ds(g, 16)]
            ti = plsc.load_gather(at_v, [iv])
            tj = plsc.load_gather(at_v, [jv])
            s = ti * ti + tj * tj
            bnd = s - jnp.where(s >= 3, 1, 0) - jnp.where(s >= 6, 2, 0)
            bond_v[pl.ds(g, 16)] = bnd.astype(jnp.float32)

        @pl.when(outwait)
        def _():
            pltpu.make_async_copy(desw, des_hbm.at[pl.ds(0, PACK)], so).wait()
            pltpu.make_async_copy(bw, b16_hbm.at[pl.ds(0, PACK)], so).wait()

        @pl.loop(0, PACK)
        def _(r):
            for k in range(8):
                e = r * 8 + k
                desw[r, pl.ds(16 * k, 16)] = gi[e, :] + gj[e, :]
                idxe = jnp.zeros((16,), jnp.int32) + e
                bw[r, pl.ds(16 * k, 16)] = plsc.load_gather(bond_v, [idxe])

        r0 = widx * PACK
        pltpu.async_copy(desw, des_hbm.at[pl.ds(r0, PACK)], so)
        pltpu.async_copy(bw, b16_hbm.at[pl.ds(r0, PACK)], so)

    fire(wid, 0)

    @pl.loop(0, WIN_PER_SUB, step=2)
    def _(k):
        fire((k + 1) * SC_NW + wid, 1)
        compute(k * SC_NW + wid, 0, outwait=k >= 2)

        @pl.when(k + 2 < WIN_PER_SUB)
        def _():
            fire((k + 2) * SC_NW + wid, 0)

        compute((k + 1) * SC_NW + wid, 1, outwait=k >= 2)

    pltpu.make_async_copy(desw0, des_hbm.at[pl.ds(0, PACK)], so0).wait()
    pltpu.make_async_copy(bw0, b16_hbm.at[pl.ds(0, PACK)], so0).wait()
    pltpu.make_async_copy(desw1, des_hbm.at[pl.ds(0, PACK)], so1).wait()
    pltpu.make_async_copy(bw1, b16_hbm.at[pl.ds(0, PACK)], so1).wait()

    @pl.when(wid < WIN_REM)
    def _():
        widx = WIN_PER_SUB * SC_NW + wid
        fire(widx, 0)
        compute(widx, 0, outwait=False)
        pltpu.make_async_copy(desw0, des_hbm.at[pl.ds(0, PACK)], so0).wait()
        pltpu.make_async_copy(bw0, b16_hbm.at[pl.ds(0, PACK)], so0).wait()


@jax.jit
def _sc_gather(node_des, i_hop, j_hop, atomtype):
    mesh = plsc.VectorSubcoreMesh(core_axis_name="c", subcore_axis_name="s")
    cp = pltpu.CompilerParams(use_tc_tiling_on_sc=False)
    if "needs_layout_passes" in pltpu.CompilerParams.__dataclass_fields__:
        cp = dataclasses.replace(cp, needs_layout_passes=False)
    dbl = lambda t: [t, t]
    run = pl.kernel(
        _sc_body,
        out_type=(
            jax.ShapeDtypeStruct((ROWS, 128), jnp.float32),
            jax.ShapeDtypeStruct((ROWS, 128), jnp.float32),
        ),
        mesh=mesh,
        scratch_types=[
            pltpu.VMEM((N,), jnp.int32),
            pltpu.VMEM((WIN,), jnp.float32),
        ] + 2 * [
            pltpu.VMEM((WIN,), jnp.int32),
            pltpu.VMEM((WIN,), jnp.int32),
            pltpu.VMEM((WIN, NMAX), jnp.float32),
            pltpu.VMEM((WIN, NMAX), jnp.float32),
            pltpu.VMEM((PACK, 128), jnp.float32),
            pltpu.VMEM((PACK, 128), jnp.float32),
        ] + 4 * [
            pltpu.SemaphoreType.DMA,
        ],
        compiler_params=cp,
    )
    return run(node_des, i_hop, j_hop, atomtype)


def _tc_body(desp_ref, drhp_ref, b16_ref, kenv_ref, kedge_ref,
             cmat_ref, benv_ref, bedge_ref, out_ref):
    desp = desp_ref[...]
    b16 = b16_ref[...]
    env_all = jnp.dot(desp, kenv_ref[...],
                      preferred_element_type=jnp.float32)
    env_sel = jnp.where(b16 == 0.0, env_all[:, 0:NCHUNK], 0.0)
    for b in range(1, NBOND):
        env_sel = env_sel + jnp.where(
            b16 == float(b), env_all[:, b * NCHUNK:(b + 1) * NCHUNK], 0.0)
    env = jnp.tanh(env_sel) + env_all[:, NBOND * NCHUNK:] + benv_ref[...]
    x = env * drhp_ref[...]
    hop_all = jnp.dot(x, kedge_ref[...],
                      preferred_element_type=jnp.float32)
    hop_sel = jnp.where(b16 == 0.0, hop_all[:, 0:NCHUNK], 0.0)
    for b in range(1, NBOND):
        hop_sel = hop_sel + jnp.where(
            b16 == float(b), hop_all[:, b * NCHUNK:(b + 1) * NCHUNK], 0.0)
    hop = jnp.tanh(hop_sel) + hop_all[:, NBOND * NCHUNK:] + bedge_ref[...]
    out_ref[...] = jnp.dot(hop, cmat_ref[...],
                           preferred_element_type=jnp.float32)


def _tc_dense(desp, drhp, b16, kenv, kedge, cmat, benv, bedge):
    full = lambda shape: pl.BlockSpec(shape, lambda i: (0, 0))
    return pl.pallas_call(
        _tc_body,
        grid=(ROWS // RB,),
        in_specs=[
            pl.BlockSpec((RB, 128), lambda i: (i, 0)),
            pl.BlockSpec((RB, 128), lambda i: (i, 0)),
            pl.BlockSpec((RB, 128), lambda i: (i, 0)),
            full((128, (NBOND + 1) * NCHUNK)),
            full((128, (NBOND + 1) * NCHUNK)),
            full((128, 80)),
            full((1, 128)),
            full((1, 128)),
        ],
        out_specs=pl.BlockSpec((RB, 80), lambda i: (i, 0)),
        out_shape=jax.ShapeDtypeStruct((ROWS, 80), jnp.float32),
    )(desp, drhp, b16, kenv, kedge, cmat, benv, bedge)


def kernel(node_des, edge_index, atomtype, des_radial_hop,
           W_env, W_env_adj, b_env, W_edge, W_edge_adj, b_edge):
    i_hop = edge_index[0]
    j_hop = edge_index[1]
    desp, b16 = _sc_gather(node_des, i_hop, j_hop, atomtype)

    eye8 = jnp.eye(8, dtype=jnp.float32)
    kron8 = lambda w: jnp.kron(eye8, w)
    kenv = jnp.concatenate(
        [kron8(W_env[b].T) for b in range(NBOND)] + [kron8(W_env_adj.T)],
        axis=1)
    wep = jnp.pad(jnp.swapaxes(W_edge, 1, 2), ((0, 0), (0, 0), (0, NMAX - NOUT)))
    kedge = jnp.concatenate(
        [kron8(wep[b]) for b in range(NBOND)]
        + [kron8(jnp.pad(W_edge_adj.T, ((0, 0), (0, NMAX - NOUT))))],
        axis=1)
    benv128 = jnp.tile(b_env, 8)[None, :]
    bedge128 = jnp.tile(jnp.pad(b_edge, (0, NMAX - NOUT)), 8)[None, :]

    drhp = des_radial_hop.reshape(ROWS, 128)
    outp = _tc_dense(desp, drhp, b16, kenv, kedge,
                     jnp.asarray(_C_NP), benv128, bedge128)
    return outp.reshape(E, NOUT)


# speedup vs baseline: 27.4456x; 1.1846x over previous
import dataclasses
import functools

import jax
import jax.numpy as jnp
import numpy as np
from jax import lax
from jax.experimental import pallas as pl
from jax.experimental.pallas import tpu as pltpu
from jax.experimental.pallas import tpu_sc as plsc

N = 10000
E = 320000
NMAX = 16
NOUT = 10
NBOND = 6

SC_NC = 2
SC_NS = 16
SC_NW = SC_NC * SC_NS
WIN = 128
NWIN = E // WIN
WIN_PER_SUB = NWIN // SC_NW
WIN_REM = NWIN - WIN_PER_SUB * SC_NW

ROWS = E // 8
RB = 800
NCHUNK = 128

_P16_NP = np.zeros((8, 128), dtype=np.float32)
for _e in range(8):
    _P16_NP[_e, _e * 16:(_e + 1) * 16] = 1.0

_C_NP = np.zeros((128, 80), dtype=np.float32)
for _e in range(8):
    for _o in range(NOUT):
        _C_NP[_e * 16 + _o, _e * 10 + _o] = 1.0


PACK = WIN // 8
SLAB = WIN_PER_SUB * WIN
TAIL0 = SC_NW * SLAB


def _sc_body(nd_hbm, ih_hbm, jh_hbm, at_hbm, des_hbm, b16_hbm,
             at_v, bond_v, sli_v, slj_v,
             gi0, gj0, desw0, bw0,
             gi1, gj1, desw1, bw1,
             sg0, sg1, so0, so1):
    wid = lax.axis_index("s") * SC_NC + lax.axis_index("c")
    pltpu.sync_copy(at_hbm, at_v)
    base = wid * SLAB
    pltpu.sync_copy(ih_hbm.at[pl.ds(base, SLAB)], sli_v)
    pltpu.sync_copy(jh_hbm.at[pl.ds(base, SLAB)], slj_v)

    bufs = ((gi0, gj0, desw0, bw0, sg0, so0),
            (gi1, gj1, desw1, bw1, sg1, so1))

    def fire(k, b):
        gi, gj, _, _, sg, _ = bufs[b]
        off = k * WIN
        pltpu.async_copy(nd_hbm.at[sli_v.at[pl.ds(off, WIN)]], gi, sg)
        pltpu.async_copy(nd_hbm.at[slj_v.at[pl.ds(off, WIN)]], gj, sg)

    def compute(k, out_row, b, outwait):
        gi, gj, desw, bw, sg, so = bufs[b]
        off = k * WIN
        pltpu.make_async_copy(nd_hbm.at[sli_v.at[pl.ds(off, WIN)]], gi, sg).wait()
        pltpu.make_async_copy(nd_hbm.at[slj_v.at[pl.ds(off, WIN)]], gj, sg).wait()

        @pl.loop(0, WIN, step=16)
        def _(g):
            iv = sli_v[pl.ds(off + g, 16)]
            jv = slj_v[pl.ds(off + g, 16)]
            ti = plsc.load_gather(at_v, [iv])
            tj = plsc.load_gather(at_v, [jv])
            s = ti * ti + tj * tj
            bnd = s - jnp.where(s >= 3, 1, 0) - jnp.where(s >= 6, 2, 0)
            bond_v[pl.ds(g, 16)] = bnd.astype(jnp.float32)

        @pl.when(outwait)
        def _():
            pltpu.make_async_copy(desw, des_hbm.at[pl.ds(0, PACK)], so).wait()
            pltpu.make_async_copy(bw, b16_hbm.at[pl.ds(0, PACK)], so).wait()

        @pl.loop(0, PACK)
        def _(r):
            for kk in range(8):
                e = r * 8 + kk
                desw[r, pl.ds(16 * kk, 16)] = gi[e, :] + gj[e, :]
                idxe = jnp.zeros((16,), jnp.int32) + e
                bw[r, pl.ds(16 * kk, 16)] = plsc.load_gather(bond_v, [idxe])

        pltpu.async_copy(desw, des_hbm.at[pl.ds(out_row, PACK)], so)
        pltpu.async_copy(bw, b16_hbm.at[pl.ds(out_row, PACK)], so)

    row0 = wid * (SLAB // 8)

    fire(0, 0)

    @pl.loop(0, WIN_PER_SUB, step=2)
    def _(k):
        fire(k + 1, 1)
        compute(k, row0 + k * PACK, 0, outwait=k >= 2)

        @pl.when(k + 2 < WIN_PER_SUB)
        def _():
            fire(k + 2, 0)

        compute(k + 1, row0 + (k + 1) * PACK, 1, outwait=k >= 2)

    pltpu.make_async_copy(desw0, des_hbm.at[pl.ds(0, PACK)], so0).wait()
    pltpu.make_async_copy(bw0, b16_hbm.at[pl.ds(0, PACK)], so0).wait()
    pltpu.make_async_copy(desw1, des_hbm.at[pl.ds(0, PACK)], so1).wait()
    pltpu.make_async_copy(bw1, b16_hbm.at[pl.ds(0, PACK)], so1).wait()

    @pl.when(wid < WIN_REM)
    def _():
        tb = TAIL0 + wid * WIN
        pltpu.sync_copy(ih_hbm.at[pl.ds(tb, WIN)], sli_v.at[pl.ds(0, WIN)])
        pltpu.sync_copy(jh_hbm.at[pl.ds(tb, WIN)], slj_v.at[pl.ds(0, WIN)])
        fire(0, 0)
        compute(0, tb // 8, 0, outwait=False)
        pltpu.make_async_copy(desw0, des_hbm.at[pl.ds(0, PACK)], so0).wait()
        pltpu.make_async_copy(bw0, b16_hbm.at[pl.ds(0, PACK)], so0).wait()


@jax.jit
def _sc_gather(node_des, i_hop, j_hop, atomtype):
    mesh = plsc.VectorSubcoreMesh(core_axis_name="c", subcore_axis_name="s")
    cp = pltpu.CompilerParams(use_tc_tiling_on_sc=False)
    if "needs_layout_passes" in pltpu.CompilerParams.__dataclass_fields__:
        cp = dataclasses.replace(cp, needs_layout_passes=False)
    dbl = lambda t: [t, t]
    run = pl.kernel(
        _sc_body,
        out_type=(
            jax.ShapeDtypeStruct((ROWS, 128), jnp.float32),
            jax.ShapeDtypeStruct((ROWS, 128), jnp.float32),
        ),
        mesh=mesh,
        scratch_types=[
            pltpu.VMEM((N,), jnp.int32),
            pltpu.VMEM((WIN,), jnp.float32),
            pltpu.VMEM((SLAB,), jnp.int32),
            pltpu.VMEM((SLAB,), jnp.int32),
        ] + 2 * [
            pltpu.VMEM((WIN, NMAX), jnp.float32),
            pltpu.VMEM((WIN, NMAX), jnp.float32),
            pltpu.VMEM((PACK, 128), jnp.float32),
            pltpu.VMEM((PACK, 128), jnp.float32),
        ] + 4 * [
            pltpu.SemaphoreType.DMA,
        ],
        compiler_params=cp,
    )
    return run(node_des, i_hop, j_hop, atomtype)


def _tc_body(desp_ref, drhp_ref, b16_ref, kenv_ref, kedge_ref,
             cmat_ref, benv_ref, bedge_ref, out_ref):
    desp = desp_ref[...]
    b16 = b16_ref[...]
    env_all = jnp.dot(desp, kenv_ref[...],
                      preferred_element_type=jnp.float32)
    env_sel = jnp.where(b16 == 0.0, env_all[:, 0:NCHUNK], 0.0)
    for b in range(1, NBOND):
        env_sel = env_sel + jnp.where(
            b16 == float(b), env_all[:, b * NCHUNK:(b + 1) * NCHUNK], 0.0)
    env = jnp.tanh(env_sel) + env_all[:, NBOND * NCHUNK:] + benv_ref[...]
    x = env * drhp_ref[...]
    hop_all = jnp.dot(x, kedge_ref[...],
                      preferred_element_type=jnp.float32)
    hop_sel = jnp.where(b16 == 0.0, hop_all[:, 0:NCHUNK], 0.0)
    for b in range(1, NBOND):
        hop_sel = hop_sel + jnp.where(
            b16 == float(b), hop_all[:, b * NCHUNK:(b + 1) * NCHUNK], 0.0)
    hop_c = jnp.dot(hop_sel, cmat_ref[...],
                    preferred_element_type=jnp.float32)
    out_ref[...] = jnp.tanh(hop_c) + hop_all[:, NBOND * NCHUNK:] + bedge_ref[...]


def _tc_dense(desp, drhp, b16, kenv, kedge, cmat, benv, bedge):
    full = lambda shape: pl.BlockSpec(shape, lambda i: (0, 0))
    return pl.pallas_call(
        _tc_body,
        grid=(ROWS // RB,),
        in_specs=[
            pl.BlockSpec((RB, 128), lambda i: (i, 0)),
            pl.BlockSpec((RB, 128), lambda i: (i, 0)),
            pl.BlockSpec((RB, 128), lambda i: (i, 0)),
            full((128, (NBOND + 1) * NCHUNK)),
            full((128, NBOND * NCHUNK + 80)),
            full((128, 80)),
            full((1, 128)),
            full((1, 80)),
        ],
        out_specs=pl.BlockSpec((RB, 80), lambda i: (i, 0)),
        out_shape=jax.ShapeDtypeStruct((ROWS, 80), jnp.float32),
    )(desp, drhp, b16, kenv, kedge, cmat, benv, bedge)


def kernel(node_des, edge_index, atomtype, des_radial_hop,
           W_env, W_env_adj, b_env, W_edge, W_edge_adj, b_edge):
    i_hop = edge_index[0]
    j_hop = edge_index[1]
    desp, b16 = _sc_gather(node_des, i_hop, j_hop, atomtype)

    eye8 = jnp.eye(8, dtype=jnp.float32)
    kron8 = lambda w: jnp.kron(eye8, w)
    kenv = jnp.concatenate(
        [kron8(W_env[b].T) for b in range(NBOND)] + [kron8(W_env_adj.T)],
        axis=1)
    wep = jnp.pad(jnp.swapaxes(W_edge, 1, 2), ((0, 0), (0, 0), (0, NMAX - NOUT)))
    kedge = jnp.concatenate(
        [kron8(wep[b]) for b in range(NBOND)] + [kron8(W_edge_adj.T)],
        axis=1)
    benv128 = jnp.tile(b_env, 8)[None, :]
    bedge80 = jnp.tile(b_edge, 8)[None, :]

    drhp = des_radial_hop.reshape(ROWS, 128)
    outp = _tc_dense(desp, drhp, b16, kenv, kedge,
                     jnp.asarray(_C_NP), benv128, bedge80)
    return outp.reshape(E, NOUT)


# speedup vs baseline: 27.8172x; 1.0135x over previous
import dataclasses
import functools

import jax
import jax.numpy as jnp
import numpy as np
from jax import lax
from jax.experimental import pallas as pl
from jax.experimental.pallas import tpu as pltpu
from jax.experimental.pallas import tpu_sc as plsc

N = 10000
E = 320000
NMAX = 16
NOUT = 10
NBOND = 6

SC_NC = 2
SC_NS = 16
SC_NW = SC_NC * SC_NS
WIN = 128
NWIN = E // WIN
WIN_PER_SUB = NWIN // SC_NW
WIN_REM = NWIN - WIN_PER_SUB * SC_NW

ROWS = E // 8
RB = 800
NCHUNK = 128

_P16_NP = np.zeros((8, 128), dtype=np.float32)
for _e in range(8):
    _P16_NP[_e, _e * 16:(_e + 1) * 16] = 1.0

_C_NP = np.zeros((128, 80), dtype=np.float32)
for _e in range(8):
    for _o in range(NOUT):
        _C_NP[_e * 16 + _o, _e * 10 + _o] = 1.0


PACK = WIN // 8
SLAB = WIN_PER_SUB * WIN
TAIL0 = SC_NW * SLAB


def _sc_body(nd_hbm, ih_hbm, jh_hbm, at_hbm, des_hbm, b16_hbm,
             at_v, bond_v, sli_v, slj_v,
             gi0, gj0, desw0, bw0,
             gi1, gj1, desw1, bw1,
             gi2, gj2, desw2, bw2,
             sg0, sg1, sg2, so0, so1, so2):
    wid = lax.axis_index("s") * SC_NC + lax.axis_index("c")
    pltpu.sync_copy(at_hbm, at_v)
    base = wid * SLAB
    pltpu.sync_copy(ih_hbm.at[pl.ds(base, SLAB)], sli_v)
    pltpu.sync_copy(jh_hbm.at[pl.ds(base, SLAB)], slj_v)

    bufs = ((gi0, gj0, desw0, bw0, sg0, so0),
            (gi1, gj1, desw1, bw1, sg1, so1),
            (gi2, gj2, desw2, bw2, sg2, so2))

    def fire(k, b):
        gi, gj, _, _, sg, _ = bufs[b]
        off = k * WIN
        pltpu.async_copy(nd_hbm.at[sli_v.at[pl.ds(off, WIN)]], gi, sg)
        pltpu.async_copy(nd_hbm.at[slj_v.at[pl.ds(off, WIN)]], gj, sg)

    def compute(k, out_row, b, outwait):
        gi, gj, desw, bw, sg, so = bufs[b]
        off = k * WIN
        pltpu.make_async_copy(nd_hbm.at[sli_v.at[pl.ds(off, WIN)]], gi, sg).wait()
        pltpu.make_async_copy(nd_hbm.at[slj_v.at[pl.ds(off, WIN)]], gj, sg).wait()

        @pl.loop(0, WIN, step=16)
        def _(g):
            iv = sli_v[pl.ds(off + g, 16)]
            jv = slj_v[pl.ds(off + g, 16)]
            ti = plsc.load_gather(at_v, [iv])
            tj = plsc.load_gather(at_v, [jv])
            s = ti * ti + tj * tj
            bnd = s - jnp.where(s >= 3, 1, 0) - jnp.where(s >= 6, 2, 0)
            bond_v[pl.ds(g, 16)] = bnd.astype(jnp.float32)

        @pl.when(outwait)
        def _():
            pltpu.make_async_copy(desw, des_hbm.at[pl.ds(0, PACK)], so).wait()
            pltpu.make_async_copy(bw, b16_hbm.at[pl.ds(0, PACK)], so).wait()

        @pl.loop(0, PACK)
        def _(r):
            for kk in range(8):
                e = r * 8 + kk
                desw[r, pl.ds(16 * kk, 16)] = gi[e, :] + gj[e, :]
                idxe = jnp.zeros((16,), jnp.int32) + e
                bw[r, pl.ds(16 * kk, 16)] = plsc.load_gather(bond_v, [idxe])

        pltpu.async_copy(desw, des_hbm.at[pl.ds(out_row, PACK)], so)
        pltpu.async_copy(bw, b16_hbm.at[pl.ds(out_row, PACK)], so)

    row0 = wid * (SLAB // 8)

    fire(0, 0)
    fire(1, 1)

    @pl.loop(0, WIN_PER_SUB, step=3)
    def _(k):
        fire(k + 2, 2)
        compute(k, row0 + k * PACK, 0, outwait=k >= 3)

        @pl.when(k + 3 < WIN_PER_SUB)
        def _():
            fire(k + 3, 0)
        compute(k + 1, row0 + (k + 1) * PACK, 1, outwait=k >= 2)

        @pl.when(k + 4 < WIN_PER_SUB)
        def _():
            fire(k + 4, 1)
        compute(k + 2, row0 + (k + 2) * PACK, 2, outwait=k >= 1)

    for desw, bw, so in ((desw0, bw0, so0), (desw1, bw1, so1), (desw2, bw2, so2)):
        pltpu.make_async_copy(desw, des_hbm.at[pl.ds(0, PACK)], so).wait()
        pltpu.make_async_copy(bw, b16_hbm.at[pl.ds(0, PACK)], so).wait()

    @pl.when(wid < WIN_REM)
    def _():
        tb = TAIL0 + wid * WIN
        pltpu.sync_copy(ih_hbm.at[pl.ds(tb, WIN)], sli_v.at[pl.ds(0, WIN)])
        pltpu.sync_copy(jh_hbm.at[pl.ds(tb, WIN)], slj_v.at[pl.ds(0, WIN)])
        fire(0, 0)
        compute(0, tb // 8, 0, outwait=False)
        pltpu.make_async_copy(desw0, des_hbm.at[pl.ds(0, PACK)], so0).wait()
        pltpu.make_async_copy(bw0, b16_hbm.at[pl.ds(0, PACK)], so0).wait()


@jax.jit
def _sc_gather(node_des, i_hop, j_hop, atomtype):
    mesh = plsc.VectorSubcoreMesh(core_axis_name="c", subcore_axis_name="s")
    cp = pltpu.CompilerParams(use_tc_tiling_on_sc=False)
    if "needs_layout_passes" in pltpu.CompilerParams.__dataclass_fields__:
        cp = dataclasses.replace(cp, needs_layout_passes=False)
    dbl = lambda t: [t, t]
    run = pl.kernel(
        _sc_body,
        out_type=(
            jax.ShapeDtypeStruct((ROWS, 128), jnp.float32),
            jax.ShapeDtypeStruct((ROWS, 128), jnp.float32),
        ),
        mesh=mesh,
        scratch_types=[
            pltpu.VMEM((N,), jnp.int32),
            pltpu.VMEM((WIN,), jnp.float32),
            pltpu.VMEM((SLAB,), jnp.int32),
            pltpu.VMEM((SLAB,), jnp.int32),
        ] + 3 * [
            pltpu.VMEM((WIN, NMAX), jnp.float32),
            pltpu.VMEM((WIN, NMAX), jnp.float32),
            pltpu.VMEM((PACK, 128), jnp.float32),
            pltpu.VMEM((PACK, 128), jnp.float32),
        ] + 6 * [
            pltpu.SemaphoreType.DMA,
        ],
        compiler_params=cp,
    )
    return run(node_des, i_hop, j_hop, atomtype)


def _tc_body(desp_ref, drhp_ref, b16_ref, kenv_ref, kedge_ref,
             cmat_ref, benv_ref, bedge_ref, out_ref):
    desp = desp_ref[...]
    b16 = b16_ref[...]
    masks = [b16 == float(b) for b in range(NBOND)]
    env_all = jnp.dot(desp.astype(jnp.bfloat16), kenv_ref[...],
                      preferred_element_type=jnp.float32)
    env_sel = jnp.where(masks[0], env_all[:, 0:NCHUNK], 0.0)
    for b in range(1, NBOND):
        env_sel = env_sel + jnp.where(
            masks[b], env_all[:, b * NCHUNK:(b + 1) * NCHUNK], 0.0)
    env = jnp.tanh(env_sel) + env_all[:, NBOND * NCHUNK:] + benv_ref[...]
    x = (env * drhp_ref[...]).astype(jnp.bfloat16)
    hop_all = jnp.dot(x, kedge_ref[...],
                      preferred_element_type=jnp.float32)
    hop_sel = jnp.where(masks[0], hop_all[:, 0:NCHUNK], 0.0)
    for b in range(1, NBOND):
        hop_sel = hop_sel + jnp.where(
            masks[b], hop_all[:, b * NCHUNK:(b + 1) * NCHUNK], 0.0)
    hop_c = jnp.dot(hop_sel.astype(jnp.bfloat16), cmat_ref[...],
                    preferred_element_type=jnp.float32)
    out_ref[...] = jnp.tanh(hop_c) + hop_all[:, NBOND * NCHUNK:] + bedge_ref[...]


def _tc_dense(desp, drhp, b16, kenv, kedge, cmat, benv, bedge):
    full = lambda shape: pl.BlockSpec(shape, lambda i: (0, 0))
    return pl.pallas_call(
        _tc_body,
        grid=(ROWS // RB,),
        in_specs=[
            pl.BlockSpec((RB, 128), lambda i: (i, 0)),
            pl.BlockSpec((RB, 128), lambda i: (i, 0)),
            pl.BlockSpec((RB, 128), lambda i: (i, 0)),
            full((128, (NBOND + 1) * NCHUNK)),
            full((128, NBOND * NCHUNK + 80)),
            full((128, 80)),
            full((1, 128)),
            full((1, 80)),
        ],
        out_specs=pl.BlockSpec((RB, 80), lambda i: (i, 0)),
        out_shape=jax.ShapeDtypeStruct((ROWS, 80), jnp.float32),
    )(desp, drhp, b16, kenv, kedge, cmat, benv, bedge)


def kernel(node_des, edge_index, atomtype, des_radial_hop,
           W_env, W_env_adj, b_env, W_edge, W_edge_adj, b_edge):
    i_hop = edge_index[0]
    j_hop = edge_index[1]
    desp, b16 = _sc_gather(node_des, i_hop, j_hop, atomtype)

    eye8 = jnp.eye(8, dtype=jnp.float32)
    kron8 = lambda w: jnp.kron(eye8, w)
    kenv = jnp.concatenate(
        [kron8(W_env[b].T) for b in range(NBOND)] + [kron8(W_env_adj.T)],
        axis=1)
    wep = jnp.pad(jnp.swapaxes(W_edge, 1, 2), ((0, 0), (0, 0), (0, NMAX - NOUT)))
    kedge = jnp.concatenate(
        [kron8(wep[b]) for b in range(NBOND)] + [kron8(W_edge_adj.T)],
        axis=1)
    benv128 = jnp.tile(b_env, 8)[None, :]
    bedge80 = jnp.tile(b_edge, 8)[None, :]

    drhp = des_radial_hop.reshape(ROWS, 128)
    outp = _tc_dense(desp, drhp, b16,
                     kenv.astype(jnp.bfloat16), kedge.astype(jnp.bfloat16),
                     jnp.asarray(_C_NP).astype(jnp.bfloat16), benv128, bedge80)
    return outp.reshape(E, NOUT)


# speedup vs baseline: 37.0718x; 1.3327x over previous
import dataclasses
import functools

import jax
import jax.numpy as jnp
import numpy as np
from jax import lax
from jax.experimental import pallas as pl
from jax.experimental.pallas import tpu as pltpu
from jax.experimental.pallas import tpu_sc as plsc

N = 10000
E = 320000
NMAX = 16
NOUT = 10
NBOND = 6

SC_NC = 2
SC_NS = 16
SC_NW = SC_NC * SC_NS
WIN = 128
NWIN = E // WIN
WIN_PER_SUB = NWIN // SC_NW
WIN_REM = NWIN - WIN_PER_SUB * SC_NW

ROWS = E // 8
RB = 800
NCHUNK = 128

_P16_NP = np.zeros((8, 128), dtype=np.float32)
for _e in range(8):
    _P16_NP[_e, _e * 16:(_e + 1) * 16] = 1.0

_C_NP = np.zeros((128, 128), dtype=np.float32)
for _e in range(8):
    for _o in range(NOUT):
        _C_NP[_e * 16 + _o, _e * 10 + _o] = 1.0


PACK = WIN // 8
SLAB = WIN_PER_SUB * WIN
TAIL0 = SC_NW * SLAB


def _sc_body(nd_hbm, ih_hbm, jh_hbm, at_hbm, des_hbm, b16_hbm,
             at_v, bond_v, sli_v, slj_v,
             gi0, gj0, desw0, bw0,
             gi1, gj1, desw1, bw1,
             gi2, gj2, desw2, bw2,
             sg0, sg1, sg2, so0, so1, so2):
    wid = lax.axis_index("s") * SC_NC + lax.axis_index("c")
    pltpu.sync_copy(at_hbm, at_v)
    base = wid * SLAB
    pltpu.sync_copy(ih_hbm.at[pl.ds(base, SLAB)], sli_v)
    pltpu.sync_copy(jh_hbm.at[pl.ds(base, SLAB)], slj_v)

    bufs = ((gi0, gj0, desw0, bw0, sg0, so0),
            (gi1, gj1, desw1, bw1, sg1, so1),
            (gi2, gj2, desw2, bw2, sg2, so2))

    def fire(k, b):
        gi, gj, _, _, sg, _ = bufs[b]
        off = k * WIN
        pltpu.async_copy(nd_hbm.at[sli_v.at[pl.ds(off, WIN)]], gi, sg)
        pltpu.async_copy(nd_hbm.at[slj_v.at[pl.ds(off, WIN)]], gj, sg)

    def compute(k, out_row, b, outwait):
        gi, gj, desw, bw, sg, so = bufs[b]
        off = k * WIN
        pltpu.make_async_copy(nd_hbm.at[sli_v.at[pl.ds(off, WIN)]], gi, sg).wait()
        pltpu.make_async_copy(nd_hbm.at[slj_v.at[pl.ds(off, WIN)]], gj, sg).wait()

        @pl.loop(0, WIN, step=16)
        def _(g):
            iv = sli_v[pl.ds(off + g, 16)]
            jv = slj_v[pl.ds(off + g, 16)]
            ti = plsc.load_gather(at_v, [iv])
            tj = plsc.load_gather(at_v, [jv])
            s = ti * ti + tj * tj
            bnd = s - jnp.where(s >= 3, 1, 0) - jnp.where(s >= 6, 2, 0)
            bond_v[pl.ds(g, 16)] = bnd.astype(jnp.float32)

        @pl.when(outwait)
        def _():
            pltpu.make_async_copy(desw, des_hbm.at[pl.ds(0, PACK)], so).wait()
            pltpu.make_async_copy(bw, b16_hbm.at[pl.ds(0, PACK)], so).wait()

        @pl.loop(0, PACK)
        def _(r):
            for kk in range(8):
                e = r * 8 + kk
                desw[r, pl.ds(16 * kk, 16)] = gi[e, :] + gj[e, :]
                idxe = jnp.zeros((16,), jnp.int32) + e
                bw[r, pl.ds(16 * kk, 16)] = plsc.load_gather(bond_v, [idxe])

        pltpu.async_copy(desw, des_hbm.at[pl.ds(out_row, PACK)], so)
        pltpu.async_copy(bw, b16_hbm.at[pl.ds(out_row, PACK)], so)

    row0 = wid * (SLAB // 8)

    fire(0, 0)
    fire(1, 1)

    @pl.loop(0, WIN_PER_SUB, step=3)
    def _(k):
        fire(k + 2, 2)
        compute(k, row0 + k * PACK, 0, outwait=k >= 3)

        @pl.when(k + 3 < WIN_PER_SUB)
        def _():
            fire(k + 3, 0)
        compute(k + 1, row0 + (k + 1) * PACK, 1, outwait=k >= 2)

        @pl.when(k + 4 < WIN_PER_SUB)
        def _():
            fire(k + 4, 1)
        compute(k + 2, row0 + (k + 2) * PACK, 2, outwait=k >= 1)

    for desw, bw, so in ((desw0, bw0, so0), (desw1, bw1, so1), (desw2, bw2, so2)):
        pltpu.make_async_copy(desw, des_hbm.at[pl.ds(0, PACK)], so).wait()
        pltpu.make_async_copy(bw, b16_hbm.at[pl.ds(0, PACK)], so).wait()

    @pl.when(wid < WIN_REM)
    def _():
        tb = TAIL0 + wid * WIN
        pltpu.sync_copy(ih_hbm.at[pl.ds(tb, WIN)], sli_v.at[pl.ds(0, WIN)])
        pltpu.sync_copy(jh_hbm.at[pl.ds(tb, WIN)], slj_v.at[pl.ds(0, WIN)])
        fire(0, 0)
        compute(0, tb // 8, 0, outwait=False)
        pltpu.make_async_copy(desw0, des_hbm.at[pl.ds(0, PACK)], so0).wait()
        pltpu.make_async_copy(bw0, b16_hbm.at[pl.ds(0, PACK)], so0).wait()


@jax.jit
def _sc_gather(node_des, i_hop, j_hop, atomtype):
    mesh = plsc.VectorSubcoreMesh(core_axis_name="c", subcore_axis_name="s")
    cp = pltpu.CompilerParams(use_tc_tiling_on_sc=False)
    if "needs_layout_passes" in pltpu.CompilerParams.__dataclass_fields__:
        cp = dataclasses.replace(cp, needs_layout_passes=False)
    dbl = lambda t: [t, t]
    run = pl.kernel(
        _sc_body,
        out_type=(
            jax.ShapeDtypeStruct((ROWS, 128), jnp.float32),
            jax.ShapeDtypeStruct((ROWS, 128), jnp.float32),
        ),
        mesh=mesh,
        scratch_types=[
            pltpu.VMEM((N,), jnp.int32),
            pltpu.VMEM((WIN,), jnp.float32),
            pltpu.VMEM((SLAB,), jnp.int32),
            pltpu.VMEM((SLAB,), jnp.int32),
        ] + 3 * [
            pltpu.VMEM((WIN, NMAX), jnp.float32),
            pltpu.VMEM((WIN, NMAX), jnp.float32),
            pltpu.VMEM((PACK, 128), jnp.float32),
            pltpu.VMEM((PACK, 128), jnp.float32),
        ] + 6 * [
            pltpu.SemaphoreType.DMA,
        ],
        compiler_params=cp,
    )
    return run(node_des, i_hop, j_hop, atomtype)


CW = 6
CGRP = WIN_PER_SUB // CW


def _sc_tr_body(hp_hbm, out_hbm,
                s0, a0, c0, s1, a1, c1, sl0, sl1, so0, so1):
    wid = lax.axis_index("s") * SC_NC + lax.axis_index("c")
    w0 = wid * WIN_PER_SUB
    lanes = lax.iota(jnp.int32, 16)
    rowpat = lanes >> 3
    colpat = (lanes & 7) * NOUT

    bufs = ((s0, a0, c0, sl0, so0), (s1, a1, c1, sl1, so1))

    def load(g, b):
        s, _, _, sl, _ = bufs[b]
        pltpu.async_copy(hp_hbm.at[pl.ds((w0 + g * CW) * PACK, CW * PACK)], s, sl)

    def conv(g, b, outwait):
        s, aa, cc, sl, so = bufs[b]
        w = w0 + g * CW
        pltpu.make_async_copy(
            hp_hbm.at[pl.ds((w0 + g * CW) * PACK, CW * PACK)], s, sl).wait()

        @pl.when(outwait)
        def _():
            pltpu.make_async_copy(aa, out_hbm.at[0, pl.ds(0, CW)], so).wait()
            pltpu.make_async_copy(
                cc, out_hbm.at[1, pl.ds(0, CW), pl.ds(0, 2)], so).wait()

        @pl.loop(0, CW)
        def _(j):
            for r in range(8):
                for gc in range(8):
                    ir = rowpat + (16 * j + 2 * gc)
                    ic = colpat + r
                    aa[j, r, pl.ds(16 * gc, 16)] = plsc.load_gather(s, [ir, ic])
            for r in range(2):
                for gc in range(8):
                    ir = rowpat + (16 * j + 2 * gc)
                    ic = colpat + (8 + r)
                    cc[j, r, pl.ds(16 * gc, 16)] = plsc.load_gather(s, [ir, ic])

        pltpu.async_copy(aa, out_hbm.at[0, pl.ds(w, CW)], so)
        pltpu.async_copy(cc, out_hbm.at[1, pl.ds(w, CW), pl.ds(0, 2)], so)

    load(0, 0)

    @pl.loop(0, CGRP - 1, step=2)
    def _(g):
        load(g + 1, 1)
        conv(g, 0, outwait=g >= 2)

        @pl.when(g + 2 < CGRP)
        def _():
            load(g + 2, 0)
        conv(g + 1, 1, outwait=g >= 1)

    conv(CGRP - 1, 0, outwait=True)
    pltpu.make_async_copy(a0, out_hbm.at[0, pl.ds(0, CW)], so0).wait()
    pltpu.make_async_copy(c0, out_hbm.at[1, pl.ds(0, CW), pl.ds(0, 2)], so0).wait()
    pltpu.make_async_copy(a1, out_hbm.at[0, pl.ds(0, CW)], so1).wait()
    pltpu.make_async_copy(c1, out_hbm.at[1, pl.ds(0, CW), pl.ds(0, 2)], so1).wait()

    @pl.when(wid < WIN_REM)
    def _():
        w = SC_NW * WIN_PER_SUB + wid
        pltpu.sync_copy(hp_hbm.at[pl.ds(w * PACK, PACK)], s0.at[pl.ds(0, PACK)])

        for r in range(8):
            for gc in range(8):
                ir = rowpat + 2 * gc
                ic = colpat + r
                a0[0, r, pl.ds(16 * gc, 16)] = plsc.load_gather(s0, [ir, ic])
        for r in range(2):
            for gc in range(8):
                ir = rowpat + 2 * gc
                ic = colpat + (8 + r)
                c0[0, r, pl.ds(16 * gc, 16)] = plsc.load_gather(s0, [ir, ic])

        pltpu.sync_copy(a0.at[pl.ds(0, 1)], out_hbm.at[0, pl.ds(w, 1)])
        pltpu.sync_copy(c0.at[pl.ds(0, 1)], out_hbm.at[1, pl.ds(w, 1), pl.ds(0, 2)])


@jax.jit
def _sc_transpose(hp):
    mesh = plsc.VectorSubcoreMesh(core_axis_name="c", subcore_axis_name="s")
    cp = pltpu.CompilerParams(use_tc_tiling_on_sc=False)
    if "needs_layout_passes" in pltpu.CompilerParams.__dataclass_fields__:
        cp = dataclasses.replace(cp, needs_layout_passes=False)
    run = pl.kernel(
        _sc_tr_body,
        out_type=jax.ShapeDtypeStruct((2, NWIN, 8, 128), jnp.float32),
        mesh=mesh,
        scratch_types=2 * [
            pltpu.VMEM((CW * PACK, 128), jnp.float32),
            pltpu.VMEM((CW, 8, 128), jnp.float32),
            pltpu.VMEM((CW, 2, 128), jnp.float32),
        ] + 4 * [
            pltpu.SemaphoreType.DMA,
        ],
        compiler_params=cp,
    )
    return run(hp)


def _tc_body(desp_ref, drhp_ref, b16_ref, kenv_ref, kedge_ref,
             cmat_ref, benv_ref, bedge_ref, out_ref):
    desp = desp_ref[...]
    b16 = b16_ref[...]
    masks = [b16 == float(b) for b in range(NBOND)]
    env_all = jnp.dot(desp.astype(jnp.bfloat16), kenv_ref[...],
                      preferred_element_type=jnp.float32)
    env_sel = jnp.where(masks[0], env_all[:, 0:NCHUNK], 0.0)
    for b in range(1, NBOND):
        env_sel = env_sel + jnp.where(
            masks[b], env_all[:, b * NCHUNK:(b + 1) * NCHUNK], 0.0)
    env = jnp.tanh(env_sel) + env_all[:, NBOND * NCHUNK:] + benv_ref[...]
    x = (env * drhp_ref[...]).astype(jnp.bfloat16)
    hop_all = jnp.dot(x, kedge_ref[...],
                      preferred_element_type=jnp.float32)
    hop_sel = jnp.where(masks[0], hop_all[:, 0:NCHUNK], 0.0)
    for b in range(1, NBOND):
        hop_sel = hop_sel + jnp.where(
            masks[b], hop_all[:, b * NCHUNK:(b + 1) * NCHUNK], 0.0)
    hop_c = jnp.dot(hop_sel.astype(jnp.bfloat16), cmat_ref[...],
                    preferred_element_type=jnp.float32)
    out_ref[...] = jnp.tanh(hop_c) + hop_all[:, NBOND * NCHUNK:] + bedge_ref[...]


def _tc_dense(desp, drhp, b16, kenv, kedge, cmat, benv, bedge):
    full = lambda shape: pl.BlockSpec(shape, lambda i: (0, 0))
    return pl.pallas_call(
        _tc_body,
        grid=(ROWS // RB,),
        in_specs=[
            pl.BlockSpec((RB, 128), lambda i: (i, 0)),
            pl.BlockSpec((RB, 128), lambda i: (i, 0)),
            pl.BlockSpec((RB, 128), lambda i: (i, 0)),
            full((128, (NBOND + 1) * NCHUNK)),
            full((128, (NBOND + 1) * NCHUNK)),
            full((128, 128)),
            full((1, 128)),
            full((1, 128)),
        ],
        out_specs=pl.BlockSpec((RB, 128), lambda i: (i, 0)),
        out_shape=jax.ShapeDtypeStruct((ROWS, 128), jnp.float32),
    )(desp, drhp, b16, kenv, kedge, cmat, benv, bedge)


def kernel(node_des, edge_index, atomtype, des_radial_hop,
           W_env, W_env_adj, b_env, W_edge, W_edge_adj, b_edge):
    i_hop = edge_index[0]
    j_hop = edge_index[1]
    desp, b16 = _sc_gather(node_des, i_hop, j_hop, atomtype)

    eye8 = jnp.eye(8, dtype=jnp.float32)
    kron8 = lambda w: jnp.kron(eye8, w)
    kenv = jnp.concatenate(
        [kron8(W_env[b].T) for b in range(NBOND)] + [kron8(W_env_adj.T)],
        axis=1)
    wep = jnp.pad(jnp.swapaxes(W_edge, 1, 2), ((0, 0), (0, 0), (0, NMAX - NOUT)))
    adjp = jnp.pad(W_edge_adj.T, ((0, 0), (0, NMAX - NOUT)))
    kedge = jnp.concatenate(
        [kron8(wep[b]) for b in range(NBOND)] + [kron8(adjp)],
        axis=1)
    benv128 = jnp.tile(b_env, 8)[None, :]
    bedge128 = jnp.tile(jnp.pad(b_edge, (0, NMAX - NOUT)), 8)[None, :]

    drhp = des_radial_hop.reshape(ROWS, 128)
    outp = _tc_dense(desp, drhp, b16,
                     kenv.astype(jnp.bfloat16), kedge.astype(jnp.bfloat16),
                     jnp.asarray(_C_NP).astype(jnp.bfloat16), benv128, bedge128)
    out4 = _sc_transpose(outp)
    hop_t16 = out4.transpose((0, 2, 1, 3)).reshape(2 * 8, E)
    return jnp.swapaxes(hop_t16[:NOUT], 0, 1)
